# baseline (speedup 1.0000x reference)
import functools

import jax
import jax.numpy as jnp
from jax import lax
from jax.experimental import pallas as pl
from jax.experimental.pallas import tpu as pltpu
from jax.experimental.pallas import tpu_sc as plsc

LANES = 16
ROW = 128
CHUNK = 2048
PAD = 8
EXP = 3 * CHUNK


@functools.lru_cache(maxsize=None)
def _make_sc_gather(n: int, emb: int):
    info = plsc.get_sparse_core_info()
    nc, ns = info.num_cores, info.num_subcores
    nw = nc * ns
    per_w = n // nw
    n_chunks = per_w // CHUNK
    assert per_w * nw == n and n_chunks * CHUNK == per_w and emb == 3

    mesh = plsc.VectorSubcoreMesh(core_axis_name="c", subcore_axis_name="s")

    @functools.partial(
        pl.kernel,
        mesh=mesh,
        compiler_params=pltpu.CompilerParams(
            use_tc_tiling_on_sc=False, needs_layout_passes=False
        ),
        out_type=jax.ShapeDtypeStruct((n * emb,), jnp.float32),
        scratch_types=[
            pltpu.VMEM((CHUNK,), jnp.int32),
            pltpu.VMEM((CHUNK, PAD), jnp.float32),
            pltpu.VMEM((EXP,), jnp.float32),
            pltpu.SemaphoreType.DMA,
        ],
    )
    def k(idx_hbm, table_hbm, out_hbm, idx_v, rows_v, outb_v, sem):
        wid = lax.axis_index("s") * nc + lax.axis_index("c")
        base = wid * per_w
        iota = lax.iota(jnp.int32, LANES)
        patterns = []
        for u in range(3):
            e = iota + 16 * u
            r = e % 3
            patterns.append(((e - r) // 3, r))

        def body(c, carry):
            i0 = base + c * CHUNK
            pltpu.sync_copy(idx_hbm.at[pl.ds(i0, CHUNK)], idx_v)
            copies = [
                pltpu.async_copy(
                    table_hbm.at[idx_v.at[pl.ds(g * ROW, ROW)]],
                    rows_v.at[pl.ds(g * ROW, ROW)],
                    sem,
                )
                for g in range(CHUNK // ROW)
            ]
            for cp in copies:
                cp.wait()
            for m in range(CHUNK // LANES):
                for u, (joff, r) in enumerate(patterns):
                    v = plsc.load_gather(rows_v, [joff + m * LANES, r])
                    outb_v[pl.ds(m * LANES * 3 + 16 * u, LANES)] = v
            pltpu.sync_copy(outb_v, out_hbm.at[pl.ds(i0 * 3, EXP)])
            return carry

        lax.fori_loop(0, n_chunks, body, 0)

    return k


def kernel(x, table):
    b, s = x.shape
    vocab, emb = table.shape
    n = b * s
    table8 = jnp.pad(table, ((0, 0), (0, PAD - emb)))
    out = _make_sc_gather(n, emb)(x.reshape(n), table8)
    return out.reshape(b, s, emb)


# speedup vs baseline: 8.4977x; 8.4977x over previous
import functools

import jax
import jax.numpy as jnp
from jax import lax
from jax.experimental import pallas as pl
from jax.experimental.pallas import tpu as pltpu
from jax.experimental.pallas import tpu_sc as plsc

LANES = 16
ROW = 128
CHUNK = 1024


@functools.lru_cache(maxsize=None)
def _make_sc_gather(n: int, vocab: int, emb: int):
    info = plsc.get_sparse_core_info()
    nc, ns = info.num_cores, info.num_subcores
    nw = nc * ns
    per_w = n // nw
    n_chunks = per_w // CHUNK
    assert per_w * nw == n and n_chunks * CHUNK == per_w and emb == 3

    mesh = plsc.VectorSubcoreMesh(core_axis_name="c", subcore_axis_name="s")

    @functools.partial(
        pl.kernel,
        mesh=mesh,
        compiler_params=pltpu.CompilerParams(
            use_tc_tiling_on_sc=False, needs_layout_passes=False
        ),
        out_type=jax.ShapeDtypeStruct((emb * n,), jnp.float32),
        scratch_types=[
            pltpu.VMEM((CHUNK,), jnp.int32),
            pltpu.VMEM((emb * CHUNK,), jnp.float32),
            pltpu.SemaphoreType.DMA,
        ],
    )
    def k(idx_hbm, table_hbm, out_hbm, idx_v, outb_v, sem):
        wid = lax.axis_index("s") * nc + lax.axis_index("c")
        base = wid * per_w

        def body(c, carry):
            i0 = base + c * CHUNK
            pltpu.sync_copy(idx_hbm.at[pl.ds(i0, CHUNK)], idx_v)
            copies = [
                pltpu.async_copy(
                    table_hbm.at[pl.ds(kk * vocab, vocab)].at[
                        idx_v.at[pl.ds(g * ROW, ROW)]
                    ],
                    outb_v.at[pl.ds(kk * CHUNK + g * ROW, ROW)],
                    sem,
                )
                for kk in range(emb)
                for g in range(CHUNK // ROW)
            ]
            for cp in copies:
                cp.wait()
            for kk in range(emb):
                pltpu.sync_copy(
                    outb_v.at[pl.ds(kk * CHUNK, CHUNK)],
                    out_hbm.at[pl.ds(kk * n + i0, CHUNK)],
                )
            return carry

        lax.fori_loop(0, n_chunks, body, 0)

    return k


def kernel(x, table):
    b, s = x.shape
    vocab, emb = table.shape
    n = b * s
    table_pl = table.T.reshape(emb * vocab)
    out = _make_sc_gather(n, vocab, emb)(x.reshape(n), table_pl)
    return out.reshape(emb, b, s).transpose(1, 2, 0)


# speedup vs baseline: 9.7665x; 1.1493x over previous
import functools

import jax
import jax.numpy as jnp
from jax import lax
from jax.experimental import pallas as pl
from jax.experimental.pallas import tpu as pltpu
from jax.experimental.pallas import tpu_sc as plsc

ROW = 128
CHUNK = 1024


@functools.lru_cache(maxsize=None)
def _make_sc_gather(n: int, vocab: int, emb: int):
    info = plsc.get_sparse_core_info()
    nc, ns = info.num_cores, info.num_subcores
    nw = nc * ns
    per_w = n // nw
    n_chunks = per_w // CHUNK
    assert per_w * nw == n and n_chunks * CHUNK == per_w and emb == 3
    assert n_chunks % 2 == 0

    mesh = plsc.VectorSubcoreMesh(core_axis_name="c", subcore_axis_name="s")

    @functools.partial(
        pl.kernel,
        mesh=mesh,
        compiler_params=pltpu.CompilerParams(
            use_tc_tiling_on_sc=False, needs_layout_passes=False
        ),
        out_type=jax.ShapeDtypeStruct((emb * n,), jnp.float32),
        scratch_types=[
            pltpu.VMEM((2, CHUNK), jnp.int32),
            pltpu.VMEM((2, emb * CHUNK), jnp.float32),
            pltpu.SemaphoreType.DMA,
            pltpu.SemaphoreType.DMA,
            pltpu.SemaphoreType.DMA,
            pltpu.SemaphoreType.DMA,
            pltpu.SemaphoreType.DMA,
            pltpu.SemaphoreType.DMA,
        ],
    )
    def k(idx_hbm, table_hbm, out_hbm, idx_v, outb_v, si0, si1, sg0, sg1,
          so0, so1):
        wid = lax.axis_index("s") * nc + lax.axis_index("c")
        base = wid * per_w
        sis = (si0, si1)
        sgs = (sg0, sg1)
        sos = (so0, so1)
        last = n_chunks - 1

        def idx_start(c, b):
            pltpu.async_copy(
                idx_hbm.at[pl.ds(base + c * CHUNK, CHUNK)],
                idx_v.at[b], sis[b],
            )

        def idx_wait(b):
            pltpu.make_async_copy(
                idx_hbm.at[pl.ds(base, CHUNK)], idx_v.at[b], sis[b]
            ).wait()

        def gathers(b):
            return [
                pltpu.async_copy(
                    table_hbm.at[pl.ds(kk * vocab, vocab)].at[
                        idx_v.at[b].at[pl.ds(g * ROW, ROW)]
                    ],
                    outb_v.at[b].at[pl.ds(kk * CHUNK + g * ROW, ROW)],
                    sgs[b],
                )
                for kk in range(emb)
                for g in range(CHUNK // ROW)
            ]

        def out_start(c, b):
            i0 = base + c * CHUNK
            for kk in range(emb):
                pltpu.async_copy(
                    outb_v.at[b].at[pl.ds(kk * CHUNK, CHUNK)],
                    out_hbm.at[pl.ds(kk * n + i0, CHUNK)],
                    sos[b],
                )

        def out_wait(b):
            pltpu.make_async_copy(
                outb_v.at[b],
                out_hbm.at[pl.ds(base * emb, emb * CHUNK)],
                sos[b],
            ).wait()

        idx_start(jnp.int32(0), 0)
        idx_start(jnp.int32(1), 1)

        def body(c2, carry):
            for b in range(2):
                c = c2 * 2 + b
                idx_wait(b)

                @pl.when(c + 2 <= last)
                def _():
                    idx_start(c + 2, b)

                @pl.when(c2 > 0)
                def _():
                    out_wait(b)
                cps = gathers(b)
                for cp in cps:
                    cp.wait()
                out_start(c, b)
            return carry

        lax.fori_loop(0, n_chunks // 2, body, 0)

        for b in range(2):
            out_wait(b)

    return k


def kernel(x, table):
    b, s = x.shape
    vocab, emb = table.shape
    n = b * s
    table_pl = table.T.reshape(emb * vocab)
    out = _make_sc_gather(n, vocab, emb)(x.reshape(n), table_pl)
    return out.reshape(emb, b, s).transpose(1, 2, 0)


# speedup vs baseline: 9.7670x; 1.0001x over previous
import functools

import jax
import jax.numpy as jnp
from jax import lax
from jax.experimental import pallas as pl
from jax.experimental.pallas import tpu as pltpu
from jax.experimental.pallas import tpu_sc as plsc

ROW = 128
CHUNK = 1024


@functools.lru_cache(maxsize=None)
def _make_sc_gather(n: int, vocab: int, emb: int):
    info = plsc.get_sparse_core_info()
    nc, ns = info.num_cores, info.num_subcores
    nw = nc * ns
    per_w = n // nw
    n_chunks = per_w // CHUNK
    assert per_w * nw == n and n_chunks * CHUNK == per_w and emb == 3
    assert n_chunks % 2 == 0

    mesh = plsc.VectorSubcoreMesh(core_axis_name="c", subcore_axis_name="s")

    @functools.partial(
        pl.kernel,
        mesh=mesh,
        compiler_params=pltpu.CompilerParams(
            use_tc_tiling_on_sc=False, needs_layout_passes=False
        ),
        out_type=jax.ShapeDtypeStruct((emb * n,), jnp.float32),
        scratch_types=[
            pltpu.VMEM((2, CHUNK), jnp.int32),
            pltpu.VMEM((2, emb * CHUNK), jnp.float32),
            pltpu.SemaphoreType.DMA,
            pltpu.SemaphoreType.DMA,
            pltpu.SemaphoreType.DMA,
            pltpu.SemaphoreType.DMA,
            pltpu.SemaphoreType.DMA,
            pltpu.SemaphoreType.DMA,
        ],
    )
    def k(idx_hbm, table_hbm, out_hbm, idx_v, outb_v, si0, si1, sg0, sg1,
          so0, so1):
        wid = lax.axis_index("s") * nc + lax.axis_index("c")
        base = wid * per_w
        sis = (si0, si1)
        sgs = (sg0, sg1)
        sos = (so0, so1)
        last = n_chunks - 1

        def idx_start(c, b):
            pltpu.async_copy(
                idx_hbm.at[pl.ds(base + c * CHUNK, CHUNK)],
                idx_v.at[b], sis[b],
            )

        def idx_wait(b):
            pltpu.make_async_copy(
                idx_hbm.at[pl.ds(base, CHUNK)], idx_v.at[b], sis[b]
            ).wait()

        def gathers(b):
            return [
                pltpu.async_copy(
                    table_hbm.at[pl.ds(kk * vocab, vocab)].at[
                        idx_v.at[b].at[pl.ds(g * ROW, ROW)]
                    ],
                    outb_v.at[b].at[pl.ds(kk * CHUNK + g * ROW, ROW)],
                    sgs[b],
                )
                for kk in range(emb)
                for g in range(CHUNK // ROW)
            ]

        def out_start(c, b):
            i0 = base + c * CHUNK
            for kk in range(emb):
                pltpu.async_copy(
                    outb_v.at[b].at[pl.ds(kk * CHUNK, CHUNK)],
                    out_hbm.at[pl.ds(kk * n + i0, CHUNK)],
                    sos[b],
                )

        def out_wait(b):
            pltpu.make_async_copy(
                outb_v.at[b],
                out_hbm.at[pl.ds(base * emb, emb * CHUNK)],
                sos[b],
            ).wait()

        idx_start(jnp.int32(0), 0)
        idx_start(jnp.int32(1), 1)

        def body(c2, carry):
            for b in range(2):
                c = c2 * 2 + b
                idx_wait(b)
                @pl.when(c2 > 0)
                def _():
                    out_wait(b)
                cps = gathers(b)
                for cp in cps:
                    cp.wait()

                @pl.when(c + 2 <= last)
                def _():
                    idx_start(c + 2, b)

                out_start(c, b)
            return carry

        lax.fori_loop(0, n_chunks // 2, body, 0)

        for b in range(2):
            out_wait(b)

    return k


def kernel(x, table):
    b, s = x.shape
    vocab, emb = table.shape
    n = b * s
    table_pl = table.T.reshape(emb * vocab)
    out = _make_sc_gather(n, vocab, emb)(x.reshape(n), table_pl)
    return out.reshape(emb, b, s).transpose(1, 2, 0)


# speedup vs baseline: 10.7886x; 1.1046x over previous
import functools

import jax
import jax.numpy as jnp
from jax import lax
from jax.experimental import pallas as pl
from jax.experimental.pallas import tpu as pltpu
from jax.experimental.pallas import tpu_sc as plsc

ROW = 128
CHUNK = 1024


@functools.lru_cache(maxsize=None)
def _make_sc_gather(n: int, vocab: int, emb: int):
    info = plsc.get_sparse_core_info()
    nc, ns = info.num_cores, info.num_subcores
    nw = nc * ns
    per_w = n // nw
    n_chunks = per_w // CHUNK
    assert per_w * nw == n and n_chunks * CHUNK == per_w and emb == 3
    assert n_chunks % 2 == 0

    mesh = plsc.VectorSubcoreMesh(core_axis_name="c", subcore_axis_name="s")

    @functools.partial(
        pl.kernel,
        mesh=mesh,
        compiler_params=pltpu.CompilerParams(
            use_tc_tiling_on_sc=False, needs_layout_passes=False
        ),
        out_type=jax.ShapeDtypeStruct((emb * n,), jnp.float32),
        scratch_types=[
            pltpu.VMEM((2, CHUNK), jnp.int32),
            pltpu.VMEM((2, emb * CHUNK), jnp.float32),
            pltpu.SemaphoreType.DMA,
            pltpu.SemaphoreType.DMA,
            pltpu.SemaphoreType.DMA,
            pltpu.SemaphoreType.DMA,
            pltpu.SemaphoreType.DMA,
            pltpu.SemaphoreType.DMA,
        ],
    )
    def k(idx_hbm, table_hbm, out_hbm, idx_v, outb_v, si0, si1, sg0, sg1,
          so0, so1):
        wid = lax.axis_index("s") * nc + lax.axis_index("c")
        base = wid * per_w
        sis = (si0, si1)
        sgs = (sg0, sg1)
        sos = (so0, so1)
        last = n_chunks - 1

        def idx_start(c, b):
            pltpu.async_copy(
                idx_hbm.at[pl.ds(base + c * CHUNK, CHUNK)],
                idx_v.at[b], sis[b],
            )

        def idx_wait(b):
            pltpu.make_async_copy(
                idx_hbm.at[pl.ds(base, CHUNK)], idx_v.at[b], sis[b]
            ).wait()

        def gathers(b):
            return [
                pltpu.async_copy(
                    table_hbm.at[pl.ds(kk * vocab, vocab)].at[
                        idx_v.at[b].at[pl.ds(g * ROW, ROW)]
                    ],
                    outb_v.at[b].at[pl.ds(kk * CHUNK + g * ROW, ROW)],
                    sgs[b],
                )
                for kk in range(emb)
                for g in range(CHUNK // ROW)
            ]

        def out_start(c, b):
            i0 = base + c * CHUNK
            for kk in range(emb):
                pltpu.async_copy(
                    outb_v.at[b].at[pl.ds(kk * CHUNK, CHUNK)],
                    out_hbm.at[pl.ds(kk * n + i0, CHUNK)],
                    sos[b],
                )

        def out_wait(b):
            pltpu.make_async_copy(
                outb_v.at[b],
                out_hbm.at[pl.ds(base * emb, emb * CHUNK)],
                sos[b],
            ).wait()

        def gather_wait(b):
            pltpu.make_async_copy(
                table_hbm.at[pl.ds(0, emb * CHUNK)], outb_v.at[b], sgs[b]
            ).wait()

        idx_start(jnp.int32(0), 0)
        idx_start(jnp.int32(1), 1)
        idx_wait(0)
        gathers(0)

        def body(c2, carry):
            for b in range(2):
                c = c2 * 2 + b
                bn = 1 - b

                @pl.when(c < last)
                def _():
                    idx_wait(bn)

                @pl.when(c >= 1)
                def _():
                    out_wait(bn)

                @pl.when(c < last)
                def _():
                    gathers(bn)
                gather_wait(b)

                @pl.when(c + 2 <= last)
                def _():
                    idx_start(c + 2, b)
                out_start(c, b)
            return carry

        lax.fori_loop(0, n_chunks // 2, body, 0)

        out_wait(1)

    return k


def kernel(x, table):
    b, s = x.shape
    vocab, emb = table.shape
    n = b * s
    table_pl = table.T.reshape(emb * vocab)
    out = _make_sc_gather(n, vocab, emb)(x.reshape(n), table_pl)
    return out.reshape(emb, b, s).transpose(1, 2, 0)


# speedup vs baseline: 10.8404x; 1.0048x over previous
import functools

import jax
import jax.numpy as jnp
from jax import lax
from jax.experimental import pallas as pl
from jax.experimental.pallas import tpu as pltpu
from jax.experimental.pallas import tpu_sc as plsc

ROW = 256
CHUNK = 2048


@functools.lru_cache(maxsize=None)
def _make_sc_gather(n: int, vocab: int, emb: int):
    info = plsc.get_sparse_core_info()
    nc, ns = info.num_cores, info.num_subcores
    nw = nc * ns
    per_w = n // nw
    n_chunks = per_w // CHUNK
    assert per_w * nw == n and n_chunks * CHUNK == per_w and emb == 3
    assert n_chunks % 2 == 0

    mesh = plsc.VectorSubcoreMesh(core_axis_name="c", subcore_axis_name="s")

    @functools.partial(
        pl.kernel,
        mesh=mesh,
        compiler_params=pltpu.CompilerParams(
            use_tc_tiling_on_sc=False, needs_layout_passes=False
        ),
        out_type=jax.ShapeDtypeStruct((emb * n,), jnp.float32),
        scratch_types=[
            pltpu.VMEM((2, CHUNK), jnp.int32),
            pltpu.VMEM((2, emb * CHUNK), jnp.float32),
            pltpu.SemaphoreType.DMA,
            pltpu.SemaphoreType.DMA,
            pltpu.SemaphoreType.DMA,
            pltpu.SemaphoreType.DMA,
            pltpu.SemaphoreType.DMA,
            pltpu.SemaphoreType.DMA,
        ],
    )
    def k(idx_hbm, table_hbm, out_hbm, idx_v, outb_v, si0, si1, sg0, sg1,
          so0, so1):
        wid = lax.axis_index("s") * nc + lax.axis_index("c")
        base = wid * per_w
        sis = (si0, si1)
        sgs = (sg0, sg1)
        sos = (so0, so1)
        last = n_chunks - 1

        def idx_start(c, b):
            pltpu.async_copy(
                idx_hbm.at[pl.ds(base + c * CHUNK, CHUNK)],
                idx_v.at[b], sis[b],
            )

        def idx_wait(b):
            pltpu.make_async_copy(
                idx_hbm.at[pl.ds(base, CHUNK)], idx_v.at[b], sis[b]
            ).wait()

        def gathers(b):
            return [
                pltpu.async_copy(
                    table_hbm.at[pl.ds(kk * vocab, vocab)].at[
                        idx_v.at[b].at[pl.ds(g * ROW, ROW)]
                    ],
                    outb_v.at[b].at[pl.ds(kk * CHUNK + g * ROW, ROW)],
                    sgs[b],
                )
                for kk in range(emb)
                for g in range(CHUNK // ROW)
            ]

        def out_start(c, b):
            i0 = base + c * CHUNK
            for kk in range(emb):
                pltpu.async_copy(
                    outb_v.at[b].at[pl.ds(kk * CHUNK, CHUNK)],
                    out_hbm.at[pl.ds(kk * n + i0, CHUNK)],
                    sos[b],
                )

        def out_wait(b):
            pltpu.make_async_copy(
                outb_v.at[b],
                out_hbm.at[pl.ds(base * emb, emb * CHUNK)],
                sos[b],
            ).wait()

        def gather_wait(b):
            pltpu.make_async_copy(
                table_hbm.at[pl.ds(0, emb * CHUNK)], outb_v.at[b], sgs[b]
            ).wait()

        idx_start(jnp.int32(0), 0)
        idx_start(jnp.int32(1), 1)
        idx_wait(0)
        gathers(0)

        def body(c2, carry):
            for b in range(2):
                c = c2 * 2 + b
                bn = 1 - b

                @pl.when(c < last)
                def _():
                    idx_wait(bn)

                @pl.when(c >= 1)
                def _():
                    out_wait(bn)

                @pl.when(c < last)
                def _():
                    gathers(bn)
                gather_wait(b)

                @pl.when(c + 2 <= last)
                def _():
                    idx_start(c + 2, b)
                out_start(c, b)
            return carry

        lax.fori_loop(0, n_chunks // 2, body, 0)

        out_wait(1)

    return k


def kernel(x, table):
    b, s = x.shape
    vocab, emb = table.shape
    n = b * s
    table_pl = table.T.reshape(emb * vocab)
    out = _make_sc_gather(n, vocab, emb)(x.reshape(n), table_pl)
    return out.reshape(emb, b, s).transpose(1, 2, 0)
